# baseline (speedup 1.0000x reference)
import functools

import jax
import jax.numpy as jnp
from jax import lax
from jax.experimental import pallas as pl
from jax.experimental.pallas import tpu as pltpu
from jax.experimental.pallas import tpu_sc as plsc

NUM_ROWS = 1_000_000
EMB = 32
OUT_DIM = 32
BATCH = 16384
SEQ = 50
B_TOTAL = BATCH * SEQ

NC = 2
NS = 16
NW = NC * NS
CHUNK = 1024
SEG = CHUNK // 4
CHUNKS_PER_S = BATCH // CHUNK
N_CHUNKS_TOTAL = SEQ * CHUNKS_PER_S
CHUNKS_PER_W = N_CHUNKS_TOTAL // NW
QH = BATCH // 4


def _make_sc_gather():
    mesh = plsc.VectorSubcoreMesh(core_axis_name="c", subcore_axis_name="s")

    @functools.partial(
        pl.kernel,
        out_type=jax.ShapeDtypeStruct((B_TOTAL, EMB), jnp.float32),
        mesh=mesh,
        scratch_types=[
            pltpu.VMEM((CHUNK,), jnp.int32),
            pltpu.VMEM((CHUNK,), jnp.int32),
            pltpu.VMEM((CHUNK, EMB), jnp.float32),
            pltpu.SemaphoreType.DMA,
        ],
        compiler_params=pltpu.CompilerParams(
            use_tc_tiling_on_sc=False, needs_layout_passes=False),
    )
    def gather_k(idx_hbm, table_hbm, out_hbm, idx_s, idx_v, rows_v, sem):
        wid = lax.axis_index("s") * NC + lax.axis_index("c")

        def chunk_body(i, carry):
            c = wid * CHUNKS_PER_W + i
            s = lax.div(c, CHUNKS_PER_S)
            k = lax.rem(c, CHUNKS_PER_S)
            src_base = s * BATCH + k * SEG
            for j in range(4):
                pltpu.sync_copy(
                    idx_hbm.at[pl.ds(src_base + j * QH, SEG)],
                    idx_s.at[pl.ds(j * SEG, SEG)])

            def reorder(t, carry2):
                lane = lax.iota(jnp.int32, 16)
                const16 = (lane & 3) * SEG + (lane >> 2)
                vals = plsc.load_gather(idx_s, [const16 + t * 4])
                idx_v[pl.ds(t * 16, 16)] = vals
                return carry2

            lax.fori_loop(0, CHUNK // 16, reorder, 0)
            pltpu.async_copy(table_hbm.at[idx_v], rows_v, sem).wait()
            pltpu.sync_copy(rows_v, out_hbm.at[pl.ds(c * CHUNK, CHUNK)])
            return carry

        lax.fori_loop(0, CHUNKS_PER_W, chunk_body, 0)

    return gather_k


_sc_gather = _make_sc_gather()

BCH = BATCH
Q = BCH // 4
ROWS128 = B_TOTAL * EMB // 128


def _detile_body(x_ref, o_ref):
    o_ref[...] = x_ref[...].reshape(64, BATCH // 8)


def _tc_detile(elemT):
    return pl.pallas_call(
        _detile_body,
        grid=(7,),
        in_specs=[pl.BlockSpec((8, BATCH), lambda i: (i, 0))],
        out_specs=pl.BlockSpec((64, BATCH // 8), lambda i: (i, 0)),
        out_shape=jax.ShapeDtypeStruct((SEQ * 8, BATCH // 8), jnp.int32),
    )(elemT)


def _linear_relu_body(x_ref, w_ref, b_ref, o_ref):
    x = x_ref[...]
    w = w_ref[...]
    bias = b_ref[...]
    for j in range(4):
        xj = x[:, j * EMB:(j + 1) * EMB]
        yj = lax.dot_general(w, xj, (((1,), (1,)), ((), ())),
                             preferred_element_type=jnp.float32)
        o_ref[0, :, j * Q:(j + 1) * Q] = jnp.maximum(yj + bias, 0.0)


def _tc_linear_relu(x128, w, b2d):
    return pl.pallas_call(
        _linear_relu_body,
        grid=(SEQ,),
        in_specs=[
            pl.BlockSpec((Q, 128), lambda s: (s, 0)),
            pl.BlockSpec((EMB, EMB), lambda s: (0, 0)),
            pl.BlockSpec((OUT_DIM, 1), lambda s: (0, 0)),
        ],
        out_specs=pl.BlockSpec((1, OUT_DIM, BCH), lambda s: (s, 0, 0)),
        out_shape=jax.ShapeDtypeStruct((SEQ, OUT_DIM, BATCH), jnp.float32),
    )(x128, w, b2d)


def kernel(element, table, W, b):
    idx = _tc_detile(element.astype(jnp.int32).T).reshape(-1)
    gathered = _sc_gather(idx, table)
    x128 = gathered.reshape(ROWS128, 128)
    yT = _tc_linear_relu(x128, W, b.reshape(OUT_DIM, 1))
    return jnp.transpose(yT, (2, 0, 1))


# speedup vs baseline: 1.0013x; 1.0013x over previous
import functools

import jax
import jax.numpy as jnp
from jax import lax
from jax.experimental import pallas as pl
from jax.experimental.pallas import tpu as pltpu
from jax.experimental.pallas import tpu_sc as plsc

NUM_ROWS = 1_000_000
EMB = 32
OUT_DIM = 32
BATCH = 16384
SEQ = 50
B_TOTAL = BATCH * SEQ

NC = 2
NS = 16
NW = NC * NS
CHUNK = 1024
SEG = CHUNK // 4
CHUNKS_PER_S = BATCH // CHUNK
N_CHUNKS_TOTAL = SEQ * CHUNKS_PER_S
CHUNKS_PER_W = N_CHUNKS_TOTAL // NW
QH = BATCH // 4


def _make_sc_gather():
    mesh = plsc.VectorSubcoreMesh(core_axis_name="c", subcore_axis_name="s")

    @functools.partial(
        pl.kernel,
        out_type=jax.ShapeDtypeStruct((B_TOTAL, EMB), jnp.float32),
        mesh=mesh,
        scratch_types=[
            pltpu.VMEM((CHUNK,), jnp.int32),
            pltpu.VMEM((CHUNK,), jnp.int32),
            pltpu.VMEM((CHUNK, EMB), jnp.float32),
            pltpu.SemaphoreType.DMA,
        ],
        compiler_params=pltpu.CompilerParams(
            use_tc_tiling_on_sc=False, needs_layout_passes=False),
    )
    def gather_k(idx_hbm, table_hbm, out_hbm, idx_s, idx_v, rows_v, sem):
        wid = lax.axis_index("s") * NC + lax.axis_index("c")

        def chunk_body(i, carry):
            c = wid * CHUNKS_PER_W + i
            s = lax.div(c, CHUNKS_PER_S)
            k = lax.rem(c, CHUNKS_PER_S)
            src_base = s * BATCH + k * SEG
            for j in range(4):
                pltpu.sync_copy(
                    idx_hbm.at[pl.ds(src_base + j * QH, SEG)],
                    idx_s.at[pl.ds(j * SEG, SEG)])

            def reorder(t, carry2):
                lane = lax.iota(jnp.int32, 16)
                const16 = (lane & 3) * SEG + (lane >> 2)
                vals = plsc.load_gather(idx_s, [const16 + t * 4])
                idx_v[pl.ds(t * 16, 16)] = vals
                return carry2

            lax.fori_loop(0, CHUNK // 16, reorder, 0)
            pltpu.async_copy(table_hbm.at[idx_v], rows_v, sem).wait()
            pltpu.sync_copy(rows_v, out_hbm.at[pl.ds(c * CHUNK, CHUNK)])
            return carry

        lax.fori_loop(0, CHUNKS_PER_W, chunk_body, 0)

    return gather_k


_sc_gather = _make_sc_gather()

BCH = BATCH
Q = BCH // 4
ROWS128 = B_TOTAL * EMB // 128


def _detile_body(x_ref, o_ref):
    o_ref[...] = x_ref[...].reshape(64, BATCH // 8)


def _tc_detile(elemT):
    return pl.pallas_call(
        _detile_body,
        grid=(7,),
        in_specs=[pl.BlockSpec((8, BATCH), lambda i: (i, 0))],
        out_specs=pl.BlockSpec((64, BATCH // 8), lambda i: (i, 0)),
        out_shape=jax.ShapeDtypeStruct((SEQ * 8, BATCH // 8), jnp.int32),
    )(elemT)


TBLK = 16384
T_GRID = -(-NUM_ROWS // TBLK)


def _table_relayout_body(x_ref, eye_ref, o_ref):
    zt = lax.dot_general(x_ref[...], eye_ref[...], (((0,), (0,)), ((), ())),
                         preferred_element_type=jnp.float32)
    o_ref[...] = zt.reshape(TBLK // 4, 128)


def _tc_table_relayout(tableT, eye):
    return pl.pallas_call(
        _table_relayout_body,
        grid=(T_GRID,),
        in_specs=[
            pl.BlockSpec((EMB, TBLK), lambda i: (0, i)),
            pl.BlockSpec((EMB, EMB), lambda i: (0, 0)),
        ],
        out_specs=pl.BlockSpec((TBLK // 4, 128), lambda i: (i, 0)),
        out_shape=jax.ShapeDtypeStruct((NUM_ROWS // 4, 128), jnp.float32),
    )(tableT, eye)


def _linear_relu_body(x_ref, w_ref, b_ref, o_ref):
    x = x_ref[...]
    w = w_ref[...]
    bias = b_ref[...]
    for j in range(4):
        xj = x[:, j * EMB:(j + 1) * EMB]
        yj = lax.dot_general(w, xj, (((1,), (1,)), ((), ())),
                             preferred_element_type=jnp.float32)
        o_ref[0, :, j * Q:(j + 1) * Q] = jnp.maximum(yj + bias, 0.0)


def _tc_linear_relu(x128, w, b2d):
    return pl.pallas_call(
        _linear_relu_body,
        grid=(SEQ,),
        in_specs=[
            pl.BlockSpec((Q, 128), lambda s: (s, 0)),
            pl.BlockSpec((EMB, EMB), lambda s: (0, 0)),
            pl.BlockSpec((OUT_DIM, 1), lambda s: (0, 0)),
        ],
        out_specs=pl.BlockSpec((1, OUT_DIM, BCH), lambda s: (s, 0, 0)),
        out_shape=jax.ShapeDtypeStruct((SEQ, OUT_DIM, BATCH), jnp.float32),
    )(x128, w, b2d)


def kernel(element, table, W, b):
    idx = _tc_detile(element.astype(jnp.int32).T).reshape(-1)
    table128 = lax.optimization_barrier(table.reshape(NUM_ROWS // 4, 128))
    table_lin = table128.reshape(NUM_ROWS, EMB)
    gathered = _sc_gather(idx, table_lin)
    x128 = gathered.reshape(ROWS128, 128)
    yT = _tc_linear_relu(x128, W, b.reshape(OUT_DIM, 1))
    return jnp.transpose(yT, (2, 0, 1))


# speedup vs baseline: 1.9570x; 1.9544x over previous
import functools

import jax
import jax.numpy as jnp
from jax import lax
from jax.experimental import pallas as pl
from jax.experimental.pallas import tpu as pltpu
from jax.experimental.pallas import tpu_sc as plsc

NUM_ROWS = 1_000_000
EMB = 32
OUT_DIM = 32
BATCH = 16384
SEQ = 50
B_TOTAL = BATCH * SEQ

NC = 2
NS = 16
NW = NC * NS
CHUNK = 1024
SEG = CHUNK // 4
CHUNKS_PER_S = BATCH // CHUNK
N_CHUNKS_TOTAL = SEQ * CHUNKS_PER_S
CHUNKS_PER_W = N_CHUNKS_TOTAL // NW
QH = BATCH // 4


def _make_sc_gather():
    mesh = plsc.VectorSubcoreMesh(core_axis_name="c", subcore_axis_name="s")

    @functools.partial(
        pl.kernel,
        out_type=jax.ShapeDtypeStruct((B_TOTAL, EMB), jnp.float32),
        mesh=mesh,
        scratch_types=[
            pltpu.VMEM((CHUNK,), jnp.int32),
            pltpu.VMEM((CHUNK,), jnp.int32),
            pltpu.VMEM((CHUNK, EMB), jnp.float32),
            pltpu.SemaphoreType.DMA,
        ],
        compiler_params=pltpu.CompilerParams(
            use_tc_tiling_on_sc=False, needs_layout_passes=False),
    )
    def gather_k(idx_hbm, table_hbm, out_hbm, idx_s, idx_v, rows_v, sem):
        wid = lax.axis_index("s") * NC + lax.axis_index("c")

        def chunk_body(i, carry):
            c = wid * CHUNKS_PER_W + i
            s = lax.div(c, CHUNKS_PER_S)
            k = lax.rem(c, CHUNKS_PER_S)
            src_base = s * BATCH + k * SEG
            for j in range(4):
                pltpu.sync_copy(
                    idx_hbm.at[pl.ds(src_base + j * QH, SEG)],
                    idx_s.at[pl.ds(j * SEG, SEG)])

            def reorder(t, carry2):
                lane = lax.iota(jnp.int32, 16)
                const16 = (lane & 3) * SEG + (lane >> 2)
                g = plsc.load_gather(idx_s, [const16 + t * 4])
                rho = (((g >> 14) << 14) + ((g & (T4 - 1)) << 2)
                       + ((g >> 12) & 3))
                idx_v[pl.ds(t * 16, 16)] = rho
                return carry2

            lax.fori_loop(0, CHUNK // 16, reorder, 0)
            pltpu.async_copy(table_hbm.at[idx_v], rows_v, sem).wait()
            pltpu.sync_copy(rows_v, out_hbm.at[pl.ds(c * CHUNK, CHUNK)])
            return carry

        lax.fori_loop(0, CHUNKS_PER_W, chunk_body, 0)

    return gather_k


_sc_gather = _make_sc_gather()

BCH = BATCH
Q = BCH // 4
ROWS128 = B_TOTAL * EMB // 128


def _detile_body(x_ref, o_ref):
    o_ref[...] = x_ref[...].reshape(64, BATCH // 8)


def _tc_detile(elemT):
    return pl.pallas_call(
        _detile_body,
        grid=(7,),
        in_specs=[pl.BlockSpec((8, BATCH), lambda i: (i, 0))],
        out_specs=pl.BlockSpec((64, BATCH // 8), lambda i: (i, 0)),
        out_shape=jax.ShapeDtypeStruct((SEQ * 8, BATCH // 8), jnp.int32),
    )(elemT)


TBLK = 16384
T4 = TBLK // 4
T_GRID = -(-NUM_ROWS // TBLK)
TAB_ROWS = T_GRID * TBLK


def _table_relayout_body(x_ref, p_ref, o_ref):
    x4 = x_ref[...].reshape(128, T4)
    o_ref[...] = lax.dot_general(x4, p_ref[...], (((0,), (0,)), ((), ())),
                                 preferred_element_type=jnp.float32)


def _tc_table_relayout(tableT, perm):
    return pl.pallas_call(
        _table_relayout_body,
        grid=(T_GRID,),
        in_specs=[
            pl.BlockSpec((EMB, TBLK), lambda i: (0, i)),
            pl.BlockSpec((128, 128), lambda i: (0, 0)),
        ],
        out_specs=pl.BlockSpec((T4, 128), lambda i: (i, 0)),
        out_shape=jax.ShapeDtypeStruct((T_GRID * T4, 128), jnp.float32),
    )(tableT, perm)


def _perm128():
    s = jnp.arange(128)
    return jax.nn.one_hot(32 * (s % 4) + s // 4, 128, dtype=jnp.float32)


def _linear_relu_body(x_ref, w_ref, b_ref, o_ref):
    x = x_ref[...]
    w = w_ref[...]
    bias = b_ref[...]
    for j in range(4):
        xj = x[:, j * EMB:(j + 1) * EMB]
        yj = lax.dot_general(w, xj, (((1,), (1,)), ((), ())),
                             preferred_element_type=jnp.float32)
        o_ref[0, :, j * Q:(j + 1) * Q] = jnp.maximum(yj + bias, 0.0)


def _tc_linear_relu(x128, w, b2d):
    return pl.pallas_call(
        _linear_relu_body,
        grid=(SEQ,),
        in_specs=[
            pl.BlockSpec((Q, 128), lambda s: (s, 0)),
            pl.BlockSpec((EMB, EMB), lambda s: (0, 0)),
            pl.BlockSpec((OUT_DIM, 1), lambda s: (0, 0)),
        ],
        out_specs=pl.BlockSpec((1, OUT_DIM, BCH), lambda s: (s, 0, 0)),
        out_shape=jax.ShapeDtypeStruct((SEQ, OUT_DIM, BATCH), jnp.float32),
    )(x128, w, b2d)


def kernel(element, table, W, b):
    idx = _tc_detile(element.astype(jnp.int32).T).reshape(-1)
    table128 = _tc_table_relayout(table.T, _perm128())
    table_lin = table128.reshape(TAB_ROWS, EMB)
    gathered = _sc_gather(idx, table_lin)
    x128 = gathered.reshape(ROWS128, 128)
    yT = _tc_linear_relu(x128, W, b.reshape(OUT_DIM, 1))
    return jnp.transpose(yT, (2, 0, 1))


# speedup vs baseline: 2.2172x; 1.1330x over previous
import functools

import jax
import jax.numpy as jnp
from jax import lax
from jax.experimental import pallas as pl
from jax.experimental.pallas import tpu as pltpu
from jax.experimental.pallas import tpu_sc as plsc

NUM_ROWS = 1_000_000
EMB = 32
OUT_DIM = 32
BATCH = 16384
SEQ = 50
B_TOTAL = BATCH * SEQ

NC = 2
NS = 16
NW = NC * NS
CHUNK = 512
SEG = CHUNK // 4
CHUNKS_PER_S = BATCH // CHUNK
N_CHUNKS_TOTAL = SEQ * CHUNKS_PER_S
CHUNKS_PER_W = N_CHUNKS_TOTAL // NW
PAIRS_PER_W = CHUNKS_PER_W // 2
QH = BATCH // 4


def _make_sc_gather():
    mesh = plsc.VectorSubcoreMesh(core_axis_name="c", subcore_axis_name="s")

    @functools.partial(
        pl.kernel,
        out_type=jax.ShapeDtypeStruct((B_TOTAL, EMB), jnp.float32),
        mesh=mesh,
        scratch_types=[
            pltpu.VMEM((CHUNK,), jnp.int32),
            pltpu.VMEM((CHUNK,), jnp.int32),
            pltpu.VMEM((CHUNK,), jnp.int32),
            pltpu.VMEM((CHUNK,), jnp.int32),
            pltpu.VMEM((CHUNK, EMB), jnp.float32),
            pltpu.VMEM((CHUNK, EMB), jnp.float32),
            pltpu.SemaphoreType.DMA,
            pltpu.SemaphoreType.DMA,
            pltpu.SemaphoreType.DMA,
            pltpu.SemaphoreType.DMA,
        ],
        compiler_params=pltpu.CompilerParams(
            use_tc_tiling_on_sc=False, needs_layout_passes=False),
    )
    def gather_k(idx_hbm, table_hbm, out_hbm, is0, is1, iv0, iv1, r0, r1,
                 ss, sg0, sg1, sw):
        wid = lax.axis_index("s") * NC + lax.axis_index("c")

        def stage_segs(c, idx_s):
            s = lax.div(c, CHUNKS_PER_S)
            k = lax.rem(c, CHUNKS_PER_S)
            src_base = s * BATCH + k * SEG
            return [pltpu.async_copy(
                idx_hbm.at[pl.ds(src_base + j * QH, SEG)],
                idx_s.at[pl.ds(j * SEG, SEG)], ss) for j in range(4)]

        def reorder_chunk(idx_s, idx_v):
            def reorder(t, carry2):
                lane = lax.iota(jnp.int32, 16)
                const16 = (lane & 3) * SEG + (lane >> 2)
                g = plsc.load_gather(idx_s, [const16 + t * 4])
                rho = (((g >> 14) << 14) + ((g & (T4 - 1)) << 2)
                       + ((g >> 12) & 3))
                idx_v[pl.ds(t * 16, 16)] = rho
                return carry2

            lax.fori_loop(0, CHUNK // 16, reorder, 0)

        def pair_body(t, carry):
            c0 = wid * CHUNKS_PER_W + t * 2
            c1 = c0 + 1
            a0 = stage_segs(c0, is0)
            a1 = stage_segs(c1, is1)
            for h in a0:
                h.wait()
            reorder_chunk(is0, iv0)
            g0 = pltpu.async_copy(table_hbm.at[iv0], r0, sg0)
            for h in a1:
                h.wait()
            reorder_chunk(is1, iv1)
            g1 = pltpu.async_copy(table_hbm.at[iv1], r1, sg1)
            g0.wait()
            w0 = pltpu.async_copy(r0, out_hbm.at[pl.ds(c0 * CHUNK, CHUNK)], sw)
            g1.wait()
            w1 = pltpu.async_copy(r1, out_hbm.at[pl.ds(c1 * CHUNK, CHUNK)], sw)
            w0.wait()
            w1.wait()
            return carry

        lax.fori_loop(0, PAIRS_PER_W, pair_body, 0)

    return gather_k


_sc_gather = _make_sc_gather()

BCH = BATCH
Q = BCH // 4
ROWS128 = B_TOTAL * EMB // 128


def _detile_body(x_ref, o_ref):
    o_ref[...] = x_ref[...].reshape(64, BATCH // 8)


def _tc_detile(elemT):
    return pl.pallas_call(
        _detile_body,
        grid=(7,),
        in_specs=[pl.BlockSpec((8, BATCH), lambda i: (i, 0))],
        out_specs=pl.BlockSpec((64, BATCH // 8), lambda i: (i, 0)),
        out_shape=jax.ShapeDtypeStruct((SEQ * 8, BATCH // 8), jnp.int32),
    )(elemT)


TBLK = 16384
T4 = TBLK // 4
T_GRID = -(-NUM_ROWS // TBLK)
TAB_ROWS = T_GRID * TBLK


def _table_relayout_body(x_ref, p_ref, o_ref):
    x4 = x_ref[...].reshape(128, T4)
    o_ref[...] = lax.dot_general(x4, p_ref[...], (((0,), (0,)), ((), ())),
                                 preferred_element_type=jnp.float32)


def _tc_table_relayout(tableT, perm):
    return pl.pallas_call(
        _table_relayout_body,
        grid=(T_GRID,),
        in_specs=[
            pl.BlockSpec((EMB, TBLK), lambda i: (0, i)),
            pl.BlockSpec((128, 128), lambda i: (0, 0)),
        ],
        out_specs=pl.BlockSpec((T4, 128), lambda i: (i, 0)),
        out_shape=jax.ShapeDtypeStruct((T_GRID * T4, 128), jnp.float32),
    )(tableT, perm)


def _perm128():
    s = jnp.arange(128)
    return jax.nn.one_hot(32 * (s % 4) + s // 4, 128, dtype=jnp.float32)


def _linear_relu_body(x_ref, w_ref, b_ref, o_ref):
    x = x_ref[...]
    w = w_ref[...]
    bias = b_ref[...]
    for j in range(4):
        xj = x[:, j * EMB:(j + 1) * EMB]
        yj = lax.dot_general(w, xj, (((1,), (1,)), ((), ())),
                             preferred_element_type=jnp.float32)
        o_ref[0, :, j * Q:(j + 1) * Q] = jnp.maximum(yj + bias, 0.0)


def _tc_linear_relu(x128, w, b2d):
    return pl.pallas_call(
        _linear_relu_body,
        grid=(SEQ,),
        in_specs=[
            pl.BlockSpec((Q, 128), lambda s: (s, 0)),
            pl.BlockSpec((EMB, EMB), lambda s: (0, 0)),
            pl.BlockSpec((OUT_DIM, 1), lambda s: (0, 0)),
        ],
        out_specs=pl.BlockSpec((1, OUT_DIM, BCH), lambda s: (s, 0, 0)),
        out_shape=jax.ShapeDtypeStruct((SEQ, OUT_DIM, BATCH), jnp.float32),
    )(x128, w, b2d)


def kernel(element, table, W, b):
    idx = _tc_detile(element.astype(jnp.int32).T).reshape(-1)
    table128 = _tc_table_relayout(table.T, _perm128())
    table_lin = table128.reshape(TAB_ROWS, EMB)
    gathered = _sc_gather(idx, table_lin)
    x128 = gathered.reshape(ROWS128, 128)
    yT = _tc_linear_relu(x128, W, b.reshape(OUT_DIM, 1))
    return jnp.transpose(yT, (2, 0, 1))


# speedup vs baseline: 2.3506x; 1.0602x over previous
import functools

import jax
import jax.numpy as jnp
from jax import lax
from jax.experimental import pallas as pl
from jax.experimental.pallas import tpu as pltpu
from jax.experimental.pallas import tpu_sc as plsc

NUM_ROWS = 1_000_000
EMB = 32
OUT_DIM = 32
BATCH = 16384
SEQ = 50
B_TOTAL = BATCH * SEQ

NC = 2
NS = 16
NW = NC * NS
CHUNK = 512
SEG = CHUNK // 4
CHUNKS_PER_S = BATCH // CHUNK
N_CHUNKS_TOTAL = SEQ * CHUNKS_PER_S
CHUNKS_PER_W = N_CHUNKS_TOTAL // NW
PAIRS_PER_W = CHUNKS_PER_W // 2
QH = BATCH // 4


def _make_sc_gather():
    mesh = plsc.VectorSubcoreMesh(core_axis_name="c", subcore_axis_name="s")

    @functools.partial(
        pl.kernel,
        out_type=jax.ShapeDtypeStruct((B_TOTAL, EMB), jnp.float32),
        mesh=mesh,
        scratch_types=[
            pltpu.VMEM((CHUNK,), jnp.int32),
            pltpu.VMEM((CHUNK,), jnp.int32),
            pltpu.VMEM((CHUNK,), jnp.int32),
            pltpu.VMEM((CHUNK,), jnp.int32),
            pltpu.VMEM((CHUNK, EMB), jnp.float32),
            pltpu.VMEM((CHUNK, EMB), jnp.float32),
            pltpu.SemaphoreType.DMA,
            pltpu.SemaphoreType.DMA,
            pltpu.SemaphoreType.DMA,
            pltpu.SemaphoreType.DMA,
        ],
        compiler_params=pltpu.CompilerParams(
            use_tc_tiling_on_sc=False, needs_layout_passes=False),
    )
    def gather_k(idx_hbm, table_hbm, out_hbm, is0, is1, iv0, iv1, r0, r1,
                 ss, sg0, sg1, sw):
        wid = lax.axis_index("s") * NC + lax.axis_index("c")

        def stage_segs(c, idx_s):
            s = lax.div(c, CHUNKS_PER_S)
            k = lax.rem(c, CHUNKS_PER_S)
            src_base = s * BATCH + k * SEG
            return [pltpu.async_copy(
                idx_hbm.at[pl.ds(src_base + j * QH, SEG)],
                idx_s.at[pl.ds(j * SEG, SEG)], ss) for j in range(4)]

        def reorder_chunk(idx_s, idx_v):
            def reorder(t, carry2):
                lane = lax.iota(jnp.int32, 16)
                const16 = (lane & 3) * SEG + (lane >> 2)
                g = plsc.load_gather(idx_s, [const16 + t * 4])
                rho = (((g >> LOG_TBLK) << LOG_TBLK)
                       + ((g & (T4 - 1)) << 2) + ((g >> LOG_T4) & 3))
                idx_v[pl.ds(t * 16, 16)] = rho
                return carry2

            lax.fori_loop(0, CHUNK // 16, reorder, 0)

        def pair_body(t, carry):
            c0 = wid * CHUNKS_PER_W + t * 2
            c1 = c0 + 1
            a0 = stage_segs(c0, is0)
            a1 = stage_segs(c1, is1)
            for h in a0:
                h.wait()
            reorder_chunk(is0, iv0)
            g0 = pltpu.async_copy(table_hbm.at[iv0], r0, sg0)
            for h in a1:
                h.wait()
            reorder_chunk(is1, iv1)
            g1 = pltpu.async_copy(table_hbm.at[iv1], r1, sg1)
            g0.wait()
            w0 = pltpu.async_copy(r0, out_hbm.at[pl.ds(c0 * CHUNK, CHUNK)], sw)
            g1.wait()
            w1 = pltpu.async_copy(r1, out_hbm.at[pl.ds(c1 * CHUNK, CHUNK)], sw)
            w0.wait()
            w1.wait()
            return carry

        lax.fori_loop(0, PAIRS_PER_W, pair_body, 0)

    return gather_k


_sc_gather = _make_sc_gather()

BCH = BATCH
Q = BCH // 4
ROWS128 = B_TOTAL * EMB // 128


def _detile_body(x_ref, o_ref):
    o_ref[...] = x_ref[...].reshape(64, BATCH // 8)


def _tc_detile(elemT):
    return pl.pallas_call(
        _detile_body,
        grid=(7,),
        in_specs=[pl.BlockSpec((8, BATCH), lambda i: (i, 0))],
        out_specs=pl.BlockSpec((64, BATCH // 8), lambda i: (i, 0)),
        out_shape=jax.ShapeDtypeStruct((SEQ * 8, BATCH // 8), jnp.int32),
    )(elemT)


TBLK = 32768
LOG_TBLK = 15
T4 = TBLK // 4
LOG_T4 = 13
T_GRID = -(-NUM_ROWS // TBLK)
TAB_ROWS = T_GRID * TBLK


def _table_relayout_body(x_ref, p_ref, o_ref):
    x4 = x_ref[...].reshape(128, T4)
    o_ref[...] = lax.dot_general(x4, p_ref[...], (((0,), (0,)), ((), ())),
                                 preferred_element_type=jnp.float32)


def _tc_table_relayout(tableT, perm):
    return pl.pallas_call(
        _table_relayout_body,
        grid=(T_GRID,),
        in_specs=[
            pl.BlockSpec((EMB, TBLK), lambda i: (0, i)),
            pl.BlockSpec((128, 128), lambda i: (0, 0)),
        ],
        out_specs=pl.BlockSpec((T4, 128), lambda i: (i, 0)),
        out_shape=jax.ShapeDtypeStruct((T_GRID * T4, 128), jnp.float32),
    )(tableT, perm)


def _perm128():
    s = jnp.arange(128)
    return jax.nn.one_hot(32 * (s % 4) + s // 4, 128, dtype=jnp.float32)


def _linear_relu_body(x_ref, w_ref, b_ref, o_ref):
    x = x_ref[...]
    w = w_ref[...]
    bias = b_ref[...]
    for j in range(4):
        xj = x[:, j * EMB:(j + 1) * EMB]
        yj = lax.dot_general(w, xj, (((1,), (1,)), ((), ())),
                             preferred_element_type=jnp.float32)
        o_ref[0, :, j * Q:(j + 1) * Q] = jnp.maximum(yj + bias, 0.0)


def _tc_linear_relu(x128, w, b2d):
    return pl.pallas_call(
        _linear_relu_body,
        grid=(SEQ,),
        in_specs=[
            pl.BlockSpec((Q, 128), lambda s: (s, 0)),
            pl.BlockSpec((EMB, EMB), lambda s: (0, 0)),
            pl.BlockSpec((OUT_DIM, 1), lambda s: (0, 0)),
        ],
        out_specs=pl.BlockSpec((1, OUT_DIM, BCH), lambda s: (s, 0, 0)),
        out_shape=jax.ShapeDtypeStruct((SEQ, OUT_DIM, BATCH), jnp.float32),
    )(x128, w, b2d)


def kernel(element, table, W, b):
    idx = _tc_detile(element.astype(jnp.int32).T).reshape(-1)
    table128 = _tc_table_relayout(table.T, _perm128())
    table_lin = table128.reshape(TAB_ROWS, EMB)
    gathered = _sc_gather(idx, table_lin)
    x128 = gathered.reshape(ROWS128, 128)
    yT = _tc_linear_relu(x128, W, b.reshape(OUT_DIM, 1))
    return jnp.transpose(yT, (2, 0, 1))


# speedup vs baseline: 2.4583x; 1.0458x over previous
import functools

import jax
import jax.numpy as jnp
from jax import lax
from jax.experimental import pallas as pl
from jax.experimental.pallas import tpu as pltpu
from jax.experimental.pallas import tpu_sc as plsc

NUM_ROWS = 1_000_000
EMB = 32
OUT_DIM = 32
BATCH = 16384
SEQ = 50
B_TOTAL = BATCH * SEQ

NC = 2
NS = 16
NW = NC * NS
CHUNK = 512
SEG = CHUNK // 4
CHUNKS_PER_S = BATCH // CHUNK
N_CHUNKS_TOTAL = SEQ * CHUNKS_PER_S
CHUNKS_PER_W = N_CHUNKS_TOTAL // NW
PAIRS_PER_W = CHUNKS_PER_W // 2
QH = BATCH // 4


def _make_sc_gather():
    mesh = plsc.VectorSubcoreMesh(core_axis_name="c", subcore_axis_name="s")

    @functools.partial(
        pl.kernel,
        out_type=jax.ShapeDtypeStruct((B_TOTAL, EMB), jnp.float32),
        mesh=mesh,
        scratch_types=[
            pltpu.VMEM((CHUNK,), jnp.int32),
            pltpu.VMEM((CHUNK,), jnp.int32),
            pltpu.VMEM((CHUNK,), jnp.int32),
            pltpu.VMEM((CHUNK,), jnp.int32),
            pltpu.VMEM((CHUNK, EMB), jnp.float32),
            pltpu.VMEM((CHUNK, EMB), jnp.float32),
            pltpu.SemaphoreType.DMA,
            pltpu.SemaphoreType.DMA,
            pltpu.SemaphoreType.DMA,
            pltpu.SemaphoreType.DMA,
        ],
        compiler_params=pltpu.CompilerParams(
            use_tc_tiling_on_sc=False, needs_layout_passes=False),
    )
    def gather_k(idx_hbm, table_hbm, out_hbm, is0, is1, iv0, iv1, r0, r1,
                 ss, sg0, sg1, sw):
        wid = lax.axis_index("s") * NC + lax.axis_index("c")

        def stage_segs(c, idx_s):
            s = lax.div(c, CHUNKS_PER_S)
            k = lax.rem(c, CHUNKS_PER_S)
            src_base = s * BATCH + k * SEG
            return [pltpu.async_copy(
                idx_hbm.at[pl.ds(src_base + j * QH, SEG)],
                idx_s.at[pl.ds(j * SEG, SEG)], ss) for j in range(4)]

        def reorder_chunk(idx_s, idx_v):
            def reorder(t, carry2):
                lane = lax.iota(jnp.int32, 16)
                const16 = (lane & 3) * SEG + (lane >> 2)
                g = plsc.load_gather(idx_s, [const16 + t * 4])
                rho = (((g >> LOG_TBLK) << LOG_TBLK)
                       + ((g & (T4 - 1)) << 2) + ((g >> LOG_T4) & 3))
                idx_v[pl.ds(t * 16, 16)] = rho
                return carry2

            lax.fori_loop(0, CHUNK // 16, reorder, 0)

        def pair_body(t, carry):
            c0 = wid * CHUNKS_PER_W + t * 2
            c1 = c0 + 1
            a0 = stage_segs(c0, is0)
            a1 = stage_segs(c1, is1)
            for h in a0:
                h.wait()
            reorder_chunk(is0, iv0)
            g0 = pltpu.async_copy(table_hbm.at[iv0], r0, sg0)
            for h in a1:
                h.wait()
            reorder_chunk(is1, iv1)
            g1 = pltpu.async_copy(table_hbm.at[iv1], r1, sg1)
            g0.wait()
            w0 = pltpu.async_copy(r0, out_hbm.at[pl.ds(c0 * CHUNK, CHUNK)], sw)
            g1.wait()
            w1 = pltpu.async_copy(r1, out_hbm.at[pl.ds(c1 * CHUNK, CHUNK)], sw)
            w0.wait()
            w1.wait()
            return carry

        lax.fori_loop(0, PAIRS_PER_W, pair_body, 0)

    return gather_k


_sc_gather = _make_sc_gather()

BCH = BATCH
Q = BCH // 4
ROWS128 = B_TOTAL * EMB // 128


def _detile_body(x_ref, o_ref):
    o_ref[...] = x_ref[...].reshape(64, BATCH // 8)


def _tc_detile(elemT):
    return pl.pallas_call(
        _detile_body,
        grid=(7,),
        in_specs=[pl.BlockSpec((8, BATCH), lambda i: (i, 0))],
        out_specs=pl.BlockSpec((64, BATCH // 8), lambda i: (i, 0)),
        out_shape=jax.ShapeDtypeStruct((SEQ * 8, BATCH // 8), jnp.int32),
    )(elemT)


TBLK = 32768
LOG_TBLK = 15
T4 = TBLK // 4
LOG_T4 = 13
T_GRID = -(-NUM_ROWS // TBLK)
TAB_ROWS = T_GRID * TBLK


def _table_relayout_body(x_ref, p_ref, o_ref):
    x4 = x_ref[...].reshape(128, T4)
    o_ref[...] = lax.dot_general(x4, p_ref[...], (((0,), (0,)), ((), ())),
                                 preferred_element_type=jnp.float32)


def _tc_table_relayout(tableT, perm):
    return pl.pallas_call(
        _table_relayout_body,
        grid=(T_GRID,),
        in_specs=[
            pl.BlockSpec((EMB, TBLK), lambda i: (0, i)),
            pl.BlockSpec((128, 128), lambda i: (0, 0)),
        ],
        out_specs=pl.BlockSpec((T4, 128), lambda i: (i, 0)),
        out_shape=jax.ShapeDtypeStruct((T_GRID * T4, 128), jnp.float32),
    )(tableT, perm)


def _perm128():
    s = jnp.arange(128)
    return jax.nn.one_hot(32 * (s % 4) + s // 4, 128, dtype=jnp.float32)


def _linear_relu_body(x_ref, w_ref, b_ref, o_ref):
    w = w_ref[...]
    bias = b_ref[...]
    for u in range(2):
        x = x_ref[u * Q:(u + 1) * Q, :]
        for j in range(4):
            xj = x[:, j * EMB:(j + 1) * EMB]
            yj = lax.dot_general(w, xj, (((1,), (1,)), ((), ())),
                                 preferred_element_type=jnp.float32)
            o_ref[u, :, j * Q:(j + 1) * Q] = jnp.maximum(yj + bias, 0.0)


def _tc_linear_relu(x128, w, b2d):
    return pl.pallas_call(
        _linear_relu_body,
        grid=(SEQ // 2,),
        in_specs=[
            pl.BlockSpec((2 * Q, 128), lambda s: (s, 0)),
            pl.BlockSpec((EMB, EMB), lambda s: (0, 0)),
            pl.BlockSpec((OUT_DIM, 1), lambda s: (0, 0)),
        ],
        out_specs=pl.BlockSpec((2, OUT_DIM, BCH), lambda s: (s, 0, 0)),
        out_shape=jax.ShapeDtypeStruct((SEQ, OUT_DIM, BATCH), jnp.float32),
    )(x128, w, b2d)


def kernel(element, table, W, b):
    idx = _tc_detile(element.astype(jnp.int32).T).reshape(-1)
    table128 = _tc_table_relayout(table.T, _perm128())
    table_lin = table128.reshape(TAB_ROWS, EMB)
    gathered = _sc_gather(idx, table_lin)
    x128 = gathered.reshape(ROWS128, 128)
    yT = _tc_linear_relu(x128, W, b.reshape(OUT_DIM, 1))
    return jnp.transpose(yT, (2, 0, 1))
